# Initial kernel scaffold; baseline (speedup 1.0000x reference)
import jax
import jax.numpy as jnp
from jax.experimental import pallas as pl


def kernel(hidden_states, W_router, wi, wo):
    raise NotImplementedError("write your pallas kernel here")


# speedup vs baseline: 11.3937x; 11.3937x over previous
import functools

import jax
import jax.numpy as jnp
from jax import lax
from jax.experimental import pallas as pl
from jax.experimental.pallas import tpu as pltpu
from jax.experimental.pallas import tpu_sc as plsc


def _router_body(nb, bt, e, x_ref, wr_ref, eidx_ref, rank_ref, counts_ref,
                 xsc_ref, run_ref):
    i = pl.program_id(0)

    @pl.when(i == 0)
    def _():
        run_ref[...] = jnp.zeros_like(run_ref)

    x = x_ref[...]
    logits = jnp.dot(x, wr_ref[...], preferred_element_type=jnp.float32)
    m = jnp.max(logits, axis=1, keepdims=True)
    s = jnp.sum(jnp.exp(logits - m), axis=1)
    top_p = 1.0 / s
    lane = jax.lax.broadcasted_iota(jnp.int32, (bt, e), 1)
    cand = jnp.where(logits == m, lane, e)
    eidx = jnp.min(cand, axis=1).astype(jnp.int32)
    eidx_ref[0, 0, :] = eidx
    xsc_ref[...] = x * top_p[:, None]

    one_hot = (eidx[:, None] == lane[0:1, :]).astype(jnp.float32)
    r = jax.lax.broadcasted_iota(jnp.int32, (bt, bt), 0)
    c = jax.lax.broadcasted_iota(jnp.int32, (bt, bt), 1)
    tri = (c < r).astype(jnp.float32)
    rank_blk = jnp.dot(tri, one_hot, preferred_element_type=jnp.float32)
    base = jnp.sum(one_hot * run_ref[...], axis=1)
    rank_tok = jnp.sum(rank_blk * one_hot, axis=1)
    rank_ref[0, 0, :] = (base + rank_tok).astype(jnp.int32)
    new_run = run_ref[...] + jnp.sum(one_hot, axis=0, keepdims=True)
    run_ref[...] = new_run

    @pl.when(i == nb - 1)
    def _():
        counts_ref[...] = jnp.broadcast_to(new_run.astype(jnp.int32),
                                           counts_ref.shape)


def _router(x, w_router, bt):
    t, d = x.shape
    e = w_router.shape[1]
    nb = t // bt
    return pl.pallas_call(
        functools.partial(_router_body, nb, bt, e),
        grid=(nb,),
        in_specs=[
            pl.BlockSpec((bt, d), lambda i: (i, 0)),
            pl.BlockSpec((d, e), lambda i: (0, 0)),
        ],
        out_specs=[
            pl.BlockSpec((1, 1, bt), lambda i: (i, 0, 0)),
            pl.BlockSpec((1, 1, bt), lambda i: (i, 0, 0)),
            pl.BlockSpec((8, e), lambda i: (0, 0)),
            pl.BlockSpec((bt, d), lambda i: (i, 0)),
        ],
        out_shape=[
            jax.ShapeDtypeStruct((nb, 1, bt), jnp.int32),
            jax.ShapeDtypeStruct((nb, 1, bt), jnp.int32),
            jax.ShapeDtypeStruct((8, e), jnp.int32),
            jax.ShapeDtypeStruct((t, d), jnp.float32),
        ],
        scratch_shapes=[pltpu.VMEM((1, e), jnp.float32)],
        compiler_params=pltpu.CompilerParams(
            dimension_semantics=("arbitrary",)),
    )(x, w_router)


def _gemm_body(bt, bo_ref, eo_ref, st_ref, en_ref, xs_ref, wi_ref, wo_ref,
               y_ref):
    g = pl.program_id(0)
    b = bo_ref[g]
    first = jnp.logical_or(g == 0, bo_ref[jnp.maximum(g - 1, 0)] != b)

    @pl.when(first)
    def _():
        y_ref[...] = jnp.zeros_like(y_ref)

    start = st_ref[g]
    end = en_ref[g]

    @pl.when(end > start)
    def _():
        rows = jax.lax.broadcasted_iota(jnp.int32, (bt, 1), 0) + b * bt
        msk = jnp.logical_and(rows >= start, rows < end)
        xb = jnp.where(msk, xs_ref[...], 0.0)
        h = jnp.maximum(
            jnp.dot(xb, wi_ref[0], preferred_element_type=jnp.float32), 0.0)
        y_ref[...] += jnp.dot(h, wo_ref[0], preferred_element_type=jnp.float32)


def _grouped_gemm(xs, wi, wo, bo, eo, st, en, bt):
    t, d = xs.shape
    e, _, f = wi.shape
    g = bo.shape[0]
    grid_spec = pltpu.PrefetchScalarGridSpec(
        num_scalar_prefetch=4,
        grid=(g,),
        in_specs=[
            pl.BlockSpec((bt, d), lambda i, bo, eo, st, en: (bo[i], 0)),
            pl.BlockSpec((1, d, f), lambda i, bo, eo, st, en: (eo[i], 0, 0)),
            pl.BlockSpec((1, f, d), lambda i, bo, eo, st, en: (eo[i], 0, 0)),
        ],
        out_specs=pl.BlockSpec((bt, d), lambda i, bo, eo, st, en: (bo[i], 0)),
    )
    return pl.pallas_call(
        functools.partial(_gemm_body, bt),
        grid_spec=grid_spec,
        out_shape=jax.ShapeDtypeStruct((t, d), jnp.float32),
        compiler_params=pltpu.CompilerParams(
            dimension_semantics=("arbitrary",)),
    )(bo, eo, st, en, xs, wi, wo)


def _make_dispatch(t, d, e, nw, nc):
    p = t // nw
    c = min(64, p)
    nch = p // c
    mesh = plsc.VectorSubcoreMesh(core_axis_name="c", subcore_axis_name="s")

    @functools.partial(
        pl.kernel, mesh=mesh,
        out_type=[
            jax.ShapeDtypeStruct((t, d), jnp.float32),
            jax.ShapeDtypeStruct((t,), jnp.int32),
        ],
        scratch_types=[
            pltpu.VMEM((e,), jnp.int32),
            pltpu.VMEM((c,), jnp.int32),
            pltpu.VMEM((c,), jnp.int32),
            pltpu.VMEM((c,), jnp.int32),
            pltpu.VMEM((c, d), jnp.float32),
            pltpu.SemaphoreType.DMA,
        ],
        compiler_params=pltpu.CompilerParams(needs_layout_passes=False),
    )
    def dispatch(eidx_hbm, rank_hbm, offs_hbm, xsc_hbm, xs_hbm, dest_hbm,
                 offs_v, ev, rv, dv, rows_v, sem):
        wid = lax.axis_index("s") * nc + lax.axis_index("c")
        base = wid * p
        pltpu.sync_copy(offs_hbm, offs_v)

        def chunk(ci, carry):
            t0 = base + ci * c
            pltpu.sync_copy(eidx_hbm.at[pl.ds(t0, c)], ev)
            pltpu.sync_copy(rank_hbm.at[pl.ds(t0, c)], rv)
            for i in range(c // 16):
                e16 = ev[pl.ds(i * 16, 16)]
                o16 = plsc.load_gather(offs_v, [e16])
                dv[pl.ds(i * 16, 16)] = o16 + rv[pl.ds(i * 16, 16)]
            pltpu.sync_copy(dv, dest_hbm.at[pl.ds(t0, c)])
            pltpu.sync_copy(xsc_hbm.at[pl.ds(t0, c)], rows_v)
            pltpu.async_copy(rows_v, xs_hbm.at[dv], sem).wait()
            return carry

        lax.fori_loop(0, nch, chunk, 0)

    return dispatch


def _make_unsort(t, d, nw, nc):
    p = t // nw
    c = min(64, p)
    nch = p // c
    mesh = plsc.VectorSubcoreMesh(core_axis_name="c", subcore_axis_name="s")

    @functools.partial(
        pl.kernel, mesh=mesh,
        out_type=jax.ShapeDtypeStruct((t, d), jnp.float32),
        scratch_types=[
            pltpu.VMEM((c,), jnp.int32),
            pltpu.VMEM((c, d), jnp.float32),
            pltpu.SemaphoreType.DMA,
        ],
        compiler_params=pltpu.CompilerParams(needs_layout_passes=False),
    )
    def unsort(ys_hbm, dest_hbm, out_hbm, dv, rows_v, sem):
        wid = lax.axis_index("s") * nc + lax.axis_index("c")
        base = wid * p

        def chunk(ci, carry):
            t0 = base + ci * c
            pltpu.sync_copy(dest_hbm.at[pl.ds(t0, c)], dv)
            pltpu.async_copy(ys_hbm.at[dv], rows_v, sem).wait()
            pltpu.sync_copy(rows_v, out_hbm.at[pl.ds(t0, c)])
            return carry

        lax.fori_loop(0, nch, chunk, 0)

    return unsort


def kernel(hidden_states, W_router, wi, wo):
    b, s, d = hidden_states.shape
    e = W_router.shape[1]
    f = wi.shape[2]
    t = b * s
    x = hidden_states.reshape(t, d)

    bt1 = 1024
    bt = 512
    nb = t // bt

    eidx3, rank3, counts8, xsc = _router(x, W_router, bt1)
    counts = counts8[0]

    ends = jnp.cumsum(counts)
    starts = ends - counts
    fb = starts // bt
    lb = (ends - 1) // bt
    tiles = jnp.where(counts > 0, lb - fb + 1, 0)
    cum = jnp.cumsum(tiles)
    total = cum[-1]
    g = nb + e - 1
    gi = jnp.arange(g, dtype=jnp.int32)
    eo = jnp.searchsorted(cum, gi, side="right").astype(jnp.int32)
    eo_c = jnp.clip(eo, 0, e - 1)
    e_pad = jnp.searchsorted(cum, total - 1, side="right").astype(jnp.int32)
    e_pad = jnp.clip(e_pad, 0, e - 1)
    valid = gi < total
    bo = fb[eo_c] + (gi - (cum[eo_c] - tiles[eo_c]))
    bo = jnp.where(valid, bo, nb - 1).astype(jnp.int32)
    eo_f = jnp.where(valid, eo_c, e_pad).astype(jnp.int32)
    st = jnp.where(valid, jnp.maximum(starts[eo_c], bo * bt), 0).astype(jnp.int32)
    en = jnp.where(valid, jnp.minimum(ends[eo_c], (bo + 1) * bt), 0).astype(jnp.int32)

    info = plsc.get_sparse_core_info()
    nc, ns = info.num_cores, info.num_subcores
    nw = nc * ns

    dispatch = _make_dispatch(t, d, e, nw, nc)
    xs, dest = dispatch(eidx3.reshape(t), rank3.reshape(t),
                        starts.astype(jnp.int32), xsc)

    ys = _grouped_gemm(xs, wi, wo, bo, eo_f, st, en, bt)

    unsort = _make_unsort(t, d, nw, nc)
    out = unsort(ys, dest)
    return out.reshape(b, s, d)


# speedup vs baseline: 13.0257x; 1.1432x over previous
import functools

import jax
import jax.numpy as jnp
from jax import lax
from jax.experimental import pallas as pl
from jax.experimental.pallas import tpu as pltpu
from jax.experimental.pallas import tpu_sc as plsc


def _router_body(nb, bt, e, x_ref, wr_ref, enc_ref, counts_ref,
                 xsc_ref, run_ref):
    i = pl.program_id(0)

    @pl.when(i == 0)
    def _():
        run_ref[...] = jnp.zeros_like(run_ref)

    x = x_ref[...]
    logits = jnp.dot(x, wr_ref[...], preferred_element_type=jnp.float32)
    m = jnp.max(logits, axis=1, keepdims=True)
    s = jnp.dot(jnp.exp(logits - m), jnp.ones((e, 1), jnp.float32),
                preferred_element_type=jnp.float32)
    top_p = 1.0 / s
    d2 = x.shape[1] // 2
    xb16 = jax.lax.bitcast_convert_type((x * top_p).astype(jnp.bfloat16),
                                        jnp.uint16)
    lo = xb16[:, :d2].astype(jnp.uint32)
    hi = xb16[:, d2:].astype(jnp.uint32)
    xsc_ref[...] = jax.lax.bitcast_convert_type(
        lo | (hi << jnp.uint32(16)), jnp.int32)

    tie = (logits == m).astype(jnp.float32)
    rl = jax.lax.broadcasted_iota(jnp.int32, (e, e), 0)
    cl = jax.lax.broadcasted_iota(jnp.int32, (e, e), 1)
    tri_u = (rl <= cl).astype(jnp.float32)
    pref = jnp.dot(tie, tri_u, preferred_element_type=jnp.float32)
    one_hot = tie * (pref == 1.0).astype(jnp.float32)
    lanes = jax.lax.broadcasted_iota(jnp.int32, (e, 1), 0).astype(jnp.float32)
    eidx = jnp.dot(one_hot, lanes, preferred_element_type=jnp.float32)

    r = jax.lax.broadcasted_iota(jnp.int32, (bt, bt), 0)
    c = jax.lax.broadcasted_iota(jnp.int32, (bt, bt), 1)
    tri_i = (c <= r).astype(jnp.float32)
    rank_incl = jnp.dot(tri_i, one_hot, preferred_element_type=jnp.float32)
    rank_blk = rank_incl - one_hot + run_ref[...]
    ones_e = jnp.ones((e, 1), jnp.float32)
    rank = jnp.dot(rank_blk * one_hot, ones_e,
                   preferred_element_type=jnp.float32,
                   precision=jax.lax.Precision.HIGHEST)
    enc_ref[...] = (rank * float(e) + eidx).astype(jnp.int32)
    new_run = run_ref[...] + rank_incl[bt - 1:bt, :]
    run_ref[...] = new_run

    @pl.when(i == nb - 1)
    def _():
        counts_ref[...] = jnp.broadcast_to(new_run.astype(jnp.int32),
                                           counts_ref.shape)


def _router(x, w_router, bt):
    t, d = x.shape
    e = w_router.shape[1]
    nb = t // bt
    return pl.pallas_call(
        functools.partial(_router_body, nb, bt, e),
        grid=(nb,),
        in_specs=[
            pl.BlockSpec((bt, d), lambda i: (i, 0)),
            pl.BlockSpec((d, e), lambda i: (0, 0)),
        ],
        out_specs=[
            pl.BlockSpec((bt, 1), lambda i: (i, 0)),
            pl.BlockSpec((8, e), lambda i: (0, 0)),
            pl.BlockSpec((bt, d // 2), lambda i: (i, 0)),
        ],
        out_shape=[
            jax.ShapeDtypeStruct((t, 1), jnp.int32),
            jax.ShapeDtypeStruct((8, e), jnp.int32),
            jax.ShapeDtypeStruct((t, d // 2), jnp.int32),
        ],
        scratch_shapes=[pltpu.VMEM((1, e), jnp.float32)],
        compiler_params=pltpu.CompilerParams(
            dimension_semantics=("arbitrary",)),
    )(x, w_router)


def _gemm_body(d2, bo_ref, eo_ref, vld_ref, xs_ref, wi_ref, wo_ref, y_ref):
    g = pl.program_id(0)

    @pl.when(vld_ref[g] > 0)
    def _():
        u = jax.lax.bitcast_convert_type(xs_ref[...], jnp.uint32)
        x_lo = jax.lax.bitcast_convert_type(u << jnp.uint32(16),
                                            jnp.float32)
        x_hi = jax.lax.bitcast_convert_type(
            u & jnp.uint32(0xFFFF0000), jnp.float32)
        h = jnp.maximum(
            jnp.dot(x_lo, wi_ref[0, :d2], preferred_element_type=jnp.float32)
            + jnp.dot(x_hi, wi_ref[0, d2:], preferred_element_type=jnp.float32),
            0.0)
        y_ref[...] = jnp.dot(h, wo_ref[0], preferred_element_type=jnp.float32)


def _grouped_gemm(xs, wi, wo, bo, eo, vld, bt):
    t_pad, d2 = xs.shape
    e, d, f = wi.shape
    g = bo.shape[0]
    grid_spec = pltpu.PrefetchScalarGridSpec(
        num_scalar_prefetch=3,
        grid=(g,),
        in_specs=[
            pl.BlockSpec((bt, d2), lambda i, bo, eo, vld: (bo[i], 0)),
            pl.BlockSpec((1, d, f), lambda i, bo, eo, vld: (eo[i], 0, 0)),
            pl.BlockSpec((1, f, d), lambda i, bo, eo, vld: (eo[i], 0, 0)),
        ],
        out_specs=pl.BlockSpec((bt, d), lambda i, bo, eo, vld: (bo[i], 0)),
    )
    return pl.pallas_call(
        functools.partial(_gemm_body, d2),
        grid_spec=grid_spec,
        out_shape=jax.ShapeDtypeStruct((t_pad, d), jnp.float32),
        compiler_params=pltpu.CompilerParams(
            dimension_semantics=("arbitrary",)),
    )(bo, eo, vld, xs, wi, wo)


def _make_dispatch(t, t_pad, d2, e, nw, nc):
    p = t // nw
    c = min(64, p)
    nch = p // c
    log2e = e.bit_length() - 1
    mesh = plsc.VectorSubcoreMesh(core_axis_name="c", subcore_axis_name="s")

    @functools.partial(
        pl.kernel, mesh=mesh,
        out_type=[
            jax.ShapeDtypeStruct((t_pad, d2), jnp.int32),
            jax.ShapeDtypeStruct((t // c, c), jnp.int32),
        ],
        scratch_types=[
            pltpu.VMEM((e,), jnp.int32),
            pltpu.VMEM((p,), jnp.int32),
            pltpu.VMEM((nch, c), jnp.int32),
            pltpu.VMEM((2, c, d2), jnp.int32),
            pltpu.SemaphoreType.DMA,
            pltpu.SemaphoreType.DMA,
            pltpu.SemaphoreType.DMA,
            pltpu.SemaphoreType.DMA,
        ],
        compiler_params=pltpu.CompilerParams(needs_layout_passes=False),
    )
    def dispatch(enc_hbm, offs_hbm, xsc_hbm, xs_hbm, dest_hbm,
                 offs_v, enc_v, dest_v, rows2, si0, si1, so0, so1):
        wid = lax.axis_index("s") * nc + lax.axis_index("c")
        base = wid * p
        pltpu.sync_copy(offs_hbm, offs_v)
        pltpu.sync_copy(enc_hbm.at[pl.ds(base, p)], enc_v)
        per_row = c // 16
        for i in range(p // 16):
            enc16 = enc_v[pl.ds(i * 16, 16)]
            e16 = jnp.bitwise_and(enc16, e - 1)
            r16 = jax.lax.shift_right_logical(enc16, log2e)
            o16 = plsc.load_gather(offs_v, [e16])
            dest_v[i // per_row, pl.ds((i % per_row) * 16, 16)] = o16 + r16
        pltpu.sync_copy(dest_v, dest_hbm.at[pl.ds(wid * nch, nch)])

        sin = (si0, si1)
        sout = (so0, so1)
        hin = [None, None]
        hout = [None, None]
        hin[0] = pltpu.async_copy(xsc_hbm.at[pl.ds(base, c)], rows2.at[0],
                                  sin[0])
        for k in range(nch):
            b = k % 2
            hin[b].wait()
            if k >= 1:
                hout[1 - b].wait()
            if k + 1 < nch:
                hin[1 - b] = pltpu.async_copy(
                    xsc_hbm.at[pl.ds(base + (k + 1) * c, c)],
                    rows2.at[1 - b], sin[1 - b])
            hout[b] = pltpu.async_copy(rows2.at[b], xs_hbm.at[dest_v.at[k]],
                                       sout[b])
        hout[(nch - 1) % 2].wait()

    return dispatch


def _make_unsort(t, t_pad, d, nw, nc):
    p = t // nw
    c = min(64, p)
    nch = p // c
    mesh = plsc.VectorSubcoreMesh(core_axis_name="c", subcore_axis_name="s")

    @functools.partial(
        pl.kernel, mesh=mesh,
        out_type=jax.ShapeDtypeStruct((t, d), jnp.float32),
        scratch_types=[
            pltpu.VMEM((nch, c), jnp.int32),
            pltpu.VMEM((2, c, d), jnp.float32),
            pltpu.SemaphoreType.DMA,
            pltpu.SemaphoreType.DMA,
            pltpu.SemaphoreType.DMA,
            pltpu.SemaphoreType.DMA,
        ],
        compiler_params=pltpu.CompilerParams(needs_layout_passes=False),
    )
    def unsort(ys_hbm, dest_hbm, out_hbm, dest_v, rows2, si0, si1, so0, so1):
        wid = lax.axis_index("s") * nc + lax.axis_index("c")
        base = wid * p
        pltpu.sync_copy(dest_hbm.at[pl.ds(wid * nch, nch)], dest_v)

        sin = (si0, si1)
        sout = (so0, so1)
        hin = [None, None]
        hout = [None, None]
        hin[0] = pltpu.async_copy(ys_hbm.at[dest_v.at[0]], rows2.at[0],
                                  sin[0])
        for k in range(nch):
            b = k % 2
            hin[b].wait()
            if k >= 1:
                hout[1 - b].wait()
            if k + 1 < nch:
                hin[1 - b] = pltpu.async_copy(
                    ys_hbm.at[dest_v.at[k + 1]], rows2.at[1 - b],
                    sin[1 - b])
            hout[b] = pltpu.async_copy(rows2.at[b],
                                       out_hbm.at[pl.ds(base + k * c, c)],
                                       sout[b])
        hout[(nch - 1) % 2].wait()

    return unsort


def kernel(hidden_states, W_router, wi, wo):
    b, s, d = hidden_states.shape
    e = W_router.shape[1]
    f = wi.shape[2]
    t = b * s
    x = hidden_states.reshape(t, d)

    bt1 = 1024
    bt = 256
    g = t // bt + e
    t_pad = g * bt

    enc2, counts8, xsc = _router(x, W_router, bt1)
    counts = counts8[0]

    nblk = (counts + (bt - 1)) // bt
    cumblk = jnp.cumsum(nblk)
    nused = cumblk[-1]
    astart = ((cumblk - nblk) * bt).astype(jnp.int32)
    gi = jnp.arange(g, dtype=jnp.int32)
    eo = jnp.searchsorted(cumblk, gi, side="right").astype(jnp.int32)
    valid = gi < nused
    eo_f = jnp.where(valid, jnp.clip(eo, 0, e - 1), e - 1).astype(jnp.int32)
    bo = jnp.where(valid, gi, g - 1).astype(jnp.int32)
    vld = valid.astype(jnp.int32)

    info = plsc.get_sparse_core_info()
    nc, ns = info.num_cores, info.num_subcores
    nw = nc * ns

    dispatch = _make_dispatch(t, t_pad, d // 2, e, nw, nc)
    xs, dest = dispatch(enc2.reshape(t), astart, xsc)

    ys = _grouped_gemm(xs, wi, wo, bo, eo_f, vld, bt)

    unsort = _make_unsort(t, t_pad, d, nw, nc)
    out = unsort(ys, dest)
    return out.reshape(b, s, d)


# speedup vs baseline: 13.0578x; 1.0025x over previous
import functools

import jax
import jax.numpy as jnp
from jax import lax
from jax.experimental import pallas as pl
from jax.experimental.pallas import tpu as pltpu
from jax.experimental.pallas import tpu_sc as plsc


def _router_body(nb, bt, e, x_ref, wr_ref, enc_ref, counts_ref,
                 xsc_ref, run_ref):
    i = pl.program_id(0)

    @pl.when(i == 0)
    def _():
        run_ref[...] = jnp.zeros_like(run_ref)

    x = x_ref[...]
    logits = jnp.dot(x, wr_ref[...], preferred_element_type=jnp.float32)
    m = jnp.max(logits, axis=1, keepdims=True)
    s = jnp.dot(jnp.exp(logits - m), jnp.ones((e, 1), jnp.float32),
                preferred_element_type=jnp.float32)
    top_p = 1.0 / s
    d2 = x.shape[1] // 2
    xb16 = jax.lax.bitcast_convert_type((x * top_p).astype(jnp.bfloat16),
                                        jnp.uint16)
    lo = xb16[:, :d2].astype(jnp.uint32)
    hi = xb16[:, d2:].astype(jnp.uint32)
    xsc_ref[...] = jax.lax.bitcast_convert_type(
        lo | (hi << jnp.uint32(16)), jnp.int32)

    tie = (logits == m).astype(jnp.float32)
    rl = jax.lax.broadcasted_iota(jnp.int32, (e, e), 0)
    cl = jax.lax.broadcasted_iota(jnp.int32, (e, e), 1)
    tri_u = (rl <= cl).astype(jnp.float32)
    pref = jnp.dot(tie, tri_u, preferred_element_type=jnp.float32)
    one_hot = tie * (pref == 1.0).astype(jnp.float32)
    lanes = jax.lax.broadcasted_iota(jnp.int32, (e, 1), 0).astype(jnp.float32)
    eidx = jnp.dot(one_hot, lanes, preferred_element_type=jnp.float32)

    r = jax.lax.broadcasted_iota(jnp.int32, (bt, bt), 0)
    c = jax.lax.broadcasted_iota(jnp.int32, (bt, bt), 1)
    tri_i = (c <= r).astype(jnp.float32)
    rank_incl = jnp.dot(tri_i, one_hot, preferred_element_type=jnp.float32)
    rank_blk = rank_incl - one_hot + run_ref[...]
    ones_e = jnp.ones((e, 1), jnp.float32)
    rank = jnp.dot(rank_blk * one_hot, ones_e,
                   preferred_element_type=jnp.float32,
                   precision=jax.lax.Precision.HIGHEST)
    enc_ref[...] = (rank * float(e) + eidx).astype(jnp.int32)
    new_run = run_ref[...] + rank_incl[bt - 1:bt, :]
    run_ref[...] = new_run

    @pl.when(i == nb - 1)
    def _():
        counts_ref[...] = jnp.broadcast_to(new_run.astype(jnp.int32),
                                           counts_ref.shape)


def _router(x, w_router, bt):
    t, d = x.shape
    e = w_router.shape[1]
    nb = t // bt
    return pl.pallas_call(
        functools.partial(_router_body, nb, bt, e),
        grid=(nb,),
        in_specs=[
            pl.BlockSpec((bt, d), lambda i: (i, 0)),
            pl.BlockSpec((d, e), lambda i: (0, 0)),
        ],
        out_specs=[
            pl.BlockSpec((bt, 1), lambda i: (i, 0)),
            pl.BlockSpec((8, e), lambda i: (0, 0)),
            pl.BlockSpec((bt, d // 2), lambda i: (i, 0)),
        ],
        out_shape=[
            jax.ShapeDtypeStruct((t, 1), jnp.int32),
            jax.ShapeDtypeStruct((8, e), jnp.int32),
            jax.ShapeDtypeStruct((t, d // 2), jnp.int32),
        ],
        scratch_shapes=[pltpu.VMEM((1, e), jnp.float32)],
        compiler_params=pltpu.CompilerParams(
            dimension_semantics=("arbitrary",)),
    )(x, w_router)


def _gemm_body(d2, bo_ref, eo_ref, vld_ref, xs_ref, wi_ref, wo_ref, y_ref):
    g = pl.program_id(0)

    @pl.when(vld_ref[g] > 0)
    def _():
        u = jax.lax.bitcast_convert_type(xs_ref[...], jnp.uint32)
        x_lo = jax.lax.bitcast_convert_type(u << jnp.uint32(16),
                                            jnp.float32)
        x_hi = jax.lax.bitcast_convert_type(
            u & jnp.uint32(0xFFFF0000), jnp.float32)
        h = jnp.maximum(
            jnp.dot(x_lo, wi_ref[0, :d2], preferred_element_type=jnp.float32)
            + jnp.dot(x_hi, wi_ref[0, d2:], preferred_element_type=jnp.float32),
            0.0)
        y_ref[...] = jnp.dot(h, wo_ref[0], preferred_element_type=jnp.float32)


def _grouped_gemm(xs, wi, wo, bo, eo, vld, bt):
    t_pad, d2 = xs.shape
    e, d, f = wi.shape
    g = bo.shape[0]
    grid_spec = pltpu.PrefetchScalarGridSpec(
        num_scalar_prefetch=3,
        grid=(g,),
        in_specs=[
            pl.BlockSpec((bt, d2), lambda i, bo, eo, vld: (bo[i], 0)),
            pl.BlockSpec((1, d, f), lambda i, bo, eo, vld: (eo[i], 0, 0)),
            pl.BlockSpec((1, f, d), lambda i, bo, eo, vld: (eo[i], 0, 0)),
        ],
        out_specs=pl.BlockSpec((bt, d), lambda i, bo, eo, vld: (bo[i], 0)),
    )
    return pl.pallas_call(
        functools.partial(_gemm_body, d2),
        grid_spec=grid_spec,
        out_shape=jax.ShapeDtypeStruct((t_pad, d), jnp.float32),
        compiler_params=pltpu.CompilerParams(
            dimension_semantics=("parallel",)),
    )(bo, eo, vld, xs, wi, wo)


def _make_dispatch(t, t_pad, d2, e, nw, nc):
    p = t // nw
    c = min(64, p)
    nch = p // c
    log2e = e.bit_length() - 1
    mesh = plsc.VectorSubcoreMesh(core_axis_name="c", subcore_axis_name="s")

    @functools.partial(
        pl.kernel, mesh=mesh,
        out_type=[
            jax.ShapeDtypeStruct((t_pad, d2), jnp.int32),
            jax.ShapeDtypeStruct((t // c, c), jnp.int32),
        ],
        scratch_types=[
            pltpu.VMEM((e,), jnp.int32),
            pltpu.VMEM((p,), jnp.int32),
            pltpu.VMEM((nch, c), jnp.int32),
            pltpu.VMEM((2, c, d2), jnp.int32),
            pltpu.SemaphoreType.DMA,
            pltpu.SemaphoreType.DMA,
            pltpu.SemaphoreType.DMA,
            pltpu.SemaphoreType.DMA,
        ],
        compiler_params=pltpu.CompilerParams(needs_layout_passes=False),
    )
    def dispatch(enc_hbm, offs_hbm, xsc_hbm, xs_hbm, dest_hbm,
                 offs_v, enc_v, dest_v, rows2, si0, si1, so0, so1):
        wid = lax.axis_index("s") * nc + lax.axis_index("c")
        base = wid * p
        pltpu.sync_copy(offs_hbm, offs_v)
        pltpu.sync_copy(enc_hbm.at[pl.ds(base, p)], enc_v)
        per_row = c // 16
        for i in range(p // 16):
            enc16 = enc_v[pl.ds(i * 16, 16)]
            e16 = jnp.bitwise_and(enc16, e - 1)
            r16 = jax.lax.shift_right_logical(enc16, log2e)
            o16 = plsc.load_gather(offs_v, [e16])
            dest_v[i // per_row, pl.ds((i % per_row) * 16, 16)] = o16 + r16
        pltpu.sync_copy(dest_v, dest_hbm.at[pl.ds(wid * nch, nch)])

        sin = (si0, si1)
        sout = (so0, so1)
        hin = [None, None]
        hout = [None, None]
        hin[0] = pltpu.async_copy(xsc_hbm.at[pl.ds(base, c)], rows2.at[0],
                                  sin[0])
        for k in range(nch):
            b = k % 2
            hin[b].wait()
            if k >= 1:
                hout[1 - b].wait()
            if k + 1 < nch:
                hin[1 - b] = pltpu.async_copy(
                    xsc_hbm.at[pl.ds(base + (k + 1) * c, c)],
                    rows2.at[1 - b], sin[1 - b])
            hout[b] = pltpu.async_copy(rows2.at[b], xs_hbm.at[dest_v.at[k]],
                                       sout[b])
        hout[(nch - 1) % 2].wait()

    return dispatch


def _make_unsort(t, t_pad, d, nw, nc):
    p = t // nw
    c = min(64, p)
    nch = p // c
    mesh = plsc.VectorSubcoreMesh(core_axis_name="c", subcore_axis_name="s")

    @functools.partial(
        pl.kernel, mesh=mesh,
        out_type=jax.ShapeDtypeStruct((t, d), jnp.float32),
        scratch_types=[
            pltpu.VMEM((nch, c), jnp.int32),
            pltpu.VMEM((2, c, d), jnp.float32),
            pltpu.SemaphoreType.DMA,
            pltpu.SemaphoreType.DMA,
            pltpu.SemaphoreType.DMA,
            pltpu.SemaphoreType.DMA,
        ],
        compiler_params=pltpu.CompilerParams(needs_layout_passes=False),
    )
    def unsort(ys_hbm, dest_hbm, out_hbm, dest_v, rows2, si0, si1, so0, so1):
        wid = lax.axis_index("s") * nc + lax.axis_index("c")
        base = wid * p
        pltpu.sync_copy(dest_hbm.at[pl.ds(wid * nch, nch)], dest_v)

        sin = (si0, si1)
        sout = (so0, so1)
        hin = [None, None]
        hout = [None, None]
        hin[0] = pltpu.async_copy(ys_hbm.at[dest_v.at[0]], rows2.at[0],
                                  sin[0])
        for k in range(nch):
            b = k % 2
            hin[b].wait()
            if k >= 1:
                hout[1 - b].wait()
            if k + 1 < nch:
                hin[1 - b] = pltpu.async_copy(
                    ys_hbm.at[dest_v.at[k + 1]], rows2.at[1 - b],
                    sin[1 - b])
            hout[b] = pltpu.async_copy(rows2.at[b],
                                       out_hbm.at[pl.ds(base + k * c, c)],
                                       sout[b])
        hout[(nch - 1) % 2].wait()

    return unsort


def kernel(hidden_states, W_router, wi, wo):
    b, s, d = hidden_states.shape
    e = W_router.shape[1]
    f = wi.shape[2]
    t = b * s
    x = hidden_states.reshape(t, d)

    bt1 = 1024
    bt = 256
    g = t // bt + e
    t_pad = g * bt

    enc2, counts8, xsc = _router(x, W_router, bt1)
    counts = counts8[0]

    nblk = (counts + (bt - 1)) // bt
    cumblk = jnp.cumsum(nblk)
    nused = cumblk[-1]
    astart = ((cumblk - nblk) * bt).astype(jnp.int32)
    gi = jnp.arange(g, dtype=jnp.int32)
    eo = jnp.searchsorted(cumblk, gi, side="right").astype(jnp.int32)
    valid = gi < nused
    eo_f = jnp.where(valid, jnp.clip(eo, 0, e - 1), e - 1).astype(jnp.int32)
    bo = jnp.where(valid, gi, g - 1).astype(jnp.int32)
    vld = valid.astype(jnp.int32)

    info = plsc.get_sparse_core_info()
    nc, ns = info.num_cores, info.num_subcores
    nw = nc * ns

    dispatch = _make_dispatch(t, t_pad, d // 2, e, nw, nc)
    xs, dest = dispatch(enc2.reshape(t), astart, xsc)

    ys = _grouped_gemm(xs, wi, wo, bo, eo_f, vld, bt)

    unsort = _make_unsort(t, t_pad, d, nw, nc)
    out = unsort(ys, dest)
    return out.reshape(b, s, d)


# speedup vs baseline: 14.8453x; 1.1369x over previous
import functools

import jax
import jax.numpy as jnp
from jax import lax
from jax.experimental import pallas as pl
from jax.experimental.pallas import tpu as pltpu
from jax.experimental.pallas import tpu_sc as plsc


def _router_body(nb, bt, e, x_ref, wr_ref, enc_ref, counts_ref,
                 xsc_ref, run_ref):
    i = pl.program_id(0)

    @pl.when(i == 0)
    def _():
        run_ref[...] = jnp.zeros_like(run_ref)

    x = x_ref[...]
    logits = jnp.dot(x, wr_ref[...], preferred_element_type=jnp.float32)
    m = jnp.max(logits, axis=1, keepdims=True)
    s = jnp.dot(jnp.exp(logits - m), jnp.ones((e, 1), jnp.float32),
                preferred_element_type=jnp.float32)
    top_p = 1.0 / s
    d2 = x.shape[1] // 2
    xb16 = jax.lax.bitcast_convert_type((x * top_p).astype(jnp.bfloat16),
                                        jnp.uint16)
    lo = xb16[:, :d2].astype(jnp.uint32)
    hi = xb16[:, d2:].astype(jnp.uint32)
    xsc_ref[...] = jax.lax.bitcast_convert_type(
        lo | (hi << jnp.uint32(16)), jnp.int32)

    tie = (logits == m).astype(jnp.float32)
    rl = jax.lax.broadcasted_iota(jnp.int32, (e, e), 0)
    cl = jax.lax.broadcasted_iota(jnp.int32, (e, e), 1)
    tri_u = (rl <= cl).astype(jnp.float32)
    pref = jnp.dot(tie, tri_u, preferred_element_type=jnp.float32)
    one_hot = tie * (pref == 1.0).astype(jnp.float32)
    lanes = jax.lax.broadcasted_iota(jnp.int32, (e, 1), 0).astype(jnp.float32)
    eidx = jnp.dot(one_hot, lanes, preferred_element_type=jnp.float32)

    r = jax.lax.broadcasted_iota(jnp.int32, (bt, bt), 0)
    c = jax.lax.broadcasted_iota(jnp.int32, (bt, bt), 1)
    tri_i = (c <= r).astype(jnp.float32)
    rank_incl = jnp.dot(tri_i, one_hot, preferred_element_type=jnp.float32)
    rank_blk = rank_incl - one_hot + run_ref[...]
    ones_e = jnp.ones((e, 1), jnp.float32)
    rank = jnp.dot(rank_blk * one_hot, ones_e,
                   preferred_element_type=jnp.float32,
                   precision=jax.lax.Precision.HIGHEST)
    enc_ref[...] = (rank * float(e) + eidx).astype(jnp.int32)
    new_run = run_ref[...] + rank_incl[bt - 1:bt, :]
    run_ref[...] = new_run

    @pl.when(i == nb - 1)
    def _():
        counts_ref[...] = jnp.broadcast_to(new_run.astype(jnp.int32),
                                           counts_ref.shape)


def _router(x, w_router, bt):
    t, d = x.shape
    e = w_router.shape[1]
    nb = t // bt
    return pl.pallas_call(
        functools.partial(_router_body, nb, bt, e),
        grid=(nb,),
        in_specs=[
            pl.BlockSpec((bt, d), lambda i: (i, 0)),
            pl.BlockSpec((d, e), lambda i: (0, 0)),
        ],
        out_specs=[
            pl.BlockSpec((bt, 1), lambda i: (i, 0)),
            pl.BlockSpec((8, e), lambda i: (0, 0)),
            pl.BlockSpec((bt, d // 2), lambda i: (i, 0)),
        ],
        out_shape=[
            jax.ShapeDtypeStruct((t, 1), jnp.int32),
            jax.ShapeDtypeStruct((8, e), jnp.int32),
            jax.ShapeDtypeStruct((t, d // 2), jnp.int32),
        ],
        scratch_shapes=[pltpu.VMEM((1, e), jnp.float32)],
        compiler_params=pltpu.CompilerParams(
            dimension_semantics=("arbitrary",)),
    )(x, w_router)


def _gemm_body(d2, bo_ref, eo_ref, vld_ref, xs_ref, wi_ref, wo_ref, y_ref):
    g = pl.program_id(0)

    @pl.when(vld_ref[g] > 0)
    def _():
        u = jax.lax.bitcast_convert_type(xs_ref[...], jnp.uint32)
        x_lo = jax.lax.bitcast_convert_type(u << jnp.uint32(16),
                                            jnp.float32)
        x_hi = jax.lax.bitcast_convert_type(
            u & jnp.uint32(0xFFFF0000), jnp.float32)
        h = jnp.maximum(
            jnp.dot(x_lo, wi_ref[0, :d2], preferred_element_type=jnp.float32)
            + jnp.dot(x_hi, wi_ref[0, d2:], preferred_element_type=jnp.float32),
            0.0)
        y_ref[...] = jnp.dot(h, wo_ref[0], preferred_element_type=jnp.float32)


def _grouped_gemm(xs, wi, wo, bo, eo, vld, bt):
    t_pad, d2 = xs.shape
    e, d, f = wi.shape
    g = bo.shape[0]
    grid_spec = pltpu.PrefetchScalarGridSpec(
        num_scalar_prefetch=3,
        grid=(g,),
        in_specs=[
            pl.BlockSpec((bt, d2), lambda i, bo, eo, vld: (bo[i], 0)),
            pl.BlockSpec((1, d, f), lambda i, bo, eo, vld: (eo[i], 0, 0)),
            pl.BlockSpec((1, f, d), lambda i, bo, eo, vld: (eo[i], 0, 0)),
        ],
        out_specs=pl.BlockSpec((bt, d), lambda i, bo, eo, vld: (bo[i], 0)),
    )
    return pl.pallas_call(
        functools.partial(_gemm_body, d2),
        grid_spec=grid_spec,
        out_shape=jax.ShapeDtypeStruct((t_pad, d), jnp.float32),
        compiler_params=pltpu.CompilerParams(
            dimension_semantics=("parallel",)),
    )(bo, eo, vld, xs, wi, wo)


def _make_dispatch(t, t_pad, d2, e, nw, nc):
    p = t // nw
    c = min(64, p)
    nch = p // c
    log2e = e.bit_length() - 1
    mesh = plsc.VectorSubcoreMesh(core_axis_name="c", subcore_axis_name="s")

    @functools.partial(
        pl.kernel, mesh=mesh,
        out_type=[
            jax.ShapeDtypeStruct((t_pad, d2), jnp.int32),
            jax.ShapeDtypeStruct((t // c, c), jnp.int32),
        ],
        scratch_types=[
            pltpu.VMEM((e,), jnp.int32),
            pltpu.VMEM((p,), jnp.int32),
            pltpu.VMEM((nch, c), jnp.int32),
            pltpu.VMEM((2, c, d2), jnp.int32),
            pltpu.SemaphoreType.DMA,
            pltpu.SemaphoreType.DMA,
            pltpu.SemaphoreType.DMA,
            pltpu.SemaphoreType.DMA,
        ],
        compiler_params=pltpu.CompilerParams(needs_layout_passes=False),
    )
    def dispatch(enc_hbm, offs_hbm, xsc_hbm, xs_hbm, dest_hbm,
                 offs_v, enc_v, dest_v, rows2, si0, si1, so0, so1):
        wid = lax.axis_index("s") * nc + lax.axis_index("c")
        base = wid * p
        pltpu.sync_copy(offs_hbm, offs_v)
        pltpu.sync_copy(enc_hbm.at[pl.ds(base, p)], enc_v)
        per_row = c // 16
        for i in range(p // 16):
            enc16 = enc_v[pl.ds(i * 16, 16)]
            e16 = jnp.bitwise_and(enc16, e - 1)
            r16 = jax.lax.shift_right_logical(enc16, log2e)
            o16 = plsc.load_gather(offs_v, [e16])
            dest_v[i // per_row, pl.ds((i % per_row) * 16, 16)] = o16 + r16
        pltpu.sync_copy(dest_v, dest_hbm.at[pl.ds(wid * nch, nch)])

        sin = (si0, si1)
        sout = (so0, so1)
        hin = [None, None]
        hout = [None, None]
        hin[0] = pltpu.async_copy(xsc_hbm.at[pl.ds(base, c)], rows2.at[0],
                                  sin[0])
        for k in range(nch):
            b = k % 2
            hin[b].wait()
            if k >= 1:
                hout[1 - b].wait()
            if k + 1 < nch:
                hin[1 - b] = pltpu.async_copy(
                    xsc_hbm.at[pl.ds(base + (k + 1) * c, c)],
                    rows2.at[1 - b], sin[1 - b])
            hout[b] = pltpu.async_copy(rows2.at[b], xs_hbm.at[dest_v.at[k]],
                                       sout[b])
        hout[(nch - 1) % 2].wait()

    return dispatch


def _make_unsort(t, t_pad, d, nw, nc):
    p = t // nw
    c = min(64, p)
    nch = p // c
    mesh = plsc.VectorSubcoreMesh(core_axis_name="c", subcore_axis_name="s")

    @functools.partial(
        pl.kernel, mesh=mesh,
        out_type=jax.ShapeDtypeStruct((t, d), jnp.float32),
        scratch_types=[
            pltpu.VMEM((nch, c), jnp.int32),
            pltpu.VMEM((2, c, d), jnp.float32),
            pltpu.SemaphoreType.DMA,
            pltpu.SemaphoreType.DMA,
            pltpu.SemaphoreType.DMA,
            pltpu.SemaphoreType.DMA,
        ],
        compiler_params=pltpu.CompilerParams(needs_layout_passes=False),
    )
    def unsort(ys_hbm, dest_hbm, out_hbm, dest_v, rows2, si0, si1, so0, so1):
        wid = lax.axis_index("s") * nc + lax.axis_index("c")
        base = wid * p
        pltpu.sync_copy(dest_hbm.at[pl.ds(wid * nch, nch)], dest_v)

        sin = (si0, si1)
        sout = (so0, so1)
        hin = [None, None]
        hout = [None, None]
        hin[0] = pltpu.async_copy(ys_hbm.at[dest_v.at[0]], rows2.at[0],
                                  sin[0])
        for k in range(nch):
            b = k % 2
            hin[b].wait()
            if k >= 1:
                hout[1 - b].wait()
            if k + 1 < nch:
                hin[1 - b] = pltpu.async_copy(
                    ys_hbm.at[dest_v.at[k + 1]], rows2.at[1 - b],
                    sin[1 - b])
            hout[b] = pltpu.async_copy(rows2.at[b],
                                       out_hbm.at[pl.ds(base + k * c, c)],
                                       sout[b])
        hout[(nch - 1) % 2].wait()

    return unsort


def kernel(hidden_states, W_router, wi, wo):
    b, s, d = hidden_states.shape
    e = W_router.shape[1]
    f = wi.shape[2]
    t = b * s
    x = hidden_states.reshape(t, d)

    bt1 = 1024
    bt = 512
    g = t // bt + e
    t_pad = g * bt

    enc2, counts8, xsc = _router(x, W_router, bt1)
    counts = counts8[0]

    nblk = (counts + (bt - 1)) // bt
    cumblk = jnp.cumsum(nblk)
    nused = cumblk[-1]
    astart = ((cumblk - nblk) * bt).astype(jnp.int32)
    gi = jnp.arange(g, dtype=jnp.int32)
    eo = jnp.searchsorted(cumblk, gi, side="right").astype(jnp.int32)
    valid = gi < nused
    eo_f = jnp.where(valid, jnp.clip(eo, 0, e - 1), e - 1).astype(jnp.int32)
    bo = jnp.where(valid, gi, g - 1).astype(jnp.int32)
    vld = valid.astype(jnp.int32)

    info = plsc.get_sparse_core_info()
    nc, ns = info.num_cores, info.num_subcores
    nw = nc * ns

    dispatch = _make_dispatch(t, t_pad, d // 2, e, nw, nc)
    xs, dest = dispatch(enc2.reshape(t), astart, xsc)

    ys = _grouped_gemm(xs, wi, wo, bo, eo_f, vld, bt)

    unsort = _make_unsort(t, t_pad, d, nw, nc)
    out = unsort(ys, dest)
    return out.reshape(b, s, d)


# speedup vs baseline: 16.0703x; 1.0825x over previous
import functools

import jax
import jax.numpy as jnp
from jax import lax
from jax.experimental import pallas as pl
from jax.experimental.pallas import tpu as pltpu
from jax.experimental.pallas import tpu_sc as plsc


def _router_body(nb, bt, e, x_ref, wr_ref, enc_ref, counts_ref,
                 xsc_ref, run_ref):
    i = pl.program_id(0)

    @pl.when(i == 0)
    def _():
        run_ref[...] = jnp.zeros_like(run_ref)

    x = x_ref[...]
    logits = jnp.dot(x, wr_ref[...], preferred_element_type=jnp.float32)
    m = jnp.max(logits, axis=1, keepdims=True)
    s = jnp.dot(jnp.exp(logits - m), jnp.ones((e, 1), jnp.float32),
                preferred_element_type=jnp.float32)
    top_p = 1.0 / s
    d2 = x.shape[1] // 2
    xb16 = jax.lax.bitcast_convert_type((x * top_p).astype(jnp.bfloat16),
                                        jnp.uint16)
    lo = xb16[:, :d2].astype(jnp.uint32)
    hi = xb16[:, d2:].astype(jnp.uint32)
    xsc_ref[...] = jax.lax.bitcast_convert_type(
        lo | (hi << jnp.uint32(16)), jnp.int32)

    tie = (logits == m).astype(jnp.float32)
    rl = jax.lax.broadcasted_iota(jnp.int32, (e, e), 0)
    cl = jax.lax.broadcasted_iota(jnp.int32, (e, e), 1)
    tri_u = (rl <= cl).astype(jnp.float32)
    pref = jnp.dot(tie, tri_u, preferred_element_type=jnp.float32)
    one_hot = tie * (pref == 1.0).astype(jnp.float32)
    lanes = jax.lax.broadcasted_iota(jnp.int32, (e, 1), 0).astype(jnp.float32)
    eidx = jnp.dot(one_hot, lanes, preferred_element_type=jnp.float32)

    r = jax.lax.broadcasted_iota(jnp.int32, (bt, bt), 0)
    c = jax.lax.broadcasted_iota(jnp.int32, (bt, bt), 1)
    tri_i = (c <= r).astype(jnp.float32)
    rank_incl = jnp.dot(tri_i, one_hot, preferred_element_type=jnp.float32)
    rank_blk = rank_incl - one_hot + run_ref[...]
    ones_e = jnp.ones((e, 1), jnp.float32)
    rank = jnp.dot(rank_blk * one_hot, ones_e,
                   preferred_element_type=jnp.float32,
                   precision=jax.lax.Precision.HIGHEST)
    enc_ref[...] = (rank * float(e) + eidx).astype(jnp.int32)
    new_run = run_ref[...] + rank_incl[bt - 1:bt, :]
    run_ref[...] = new_run

    @pl.when(i == nb - 1)
    def _():
        counts_ref[...] = jnp.broadcast_to(new_run.astype(jnp.int32),
                                           counts_ref.shape)


def _router(x, w_router, bt):
    t, d = x.shape
    e = w_router.shape[1]
    nb = t // bt
    return pl.pallas_call(
        functools.partial(_router_body, nb, bt, e),
        grid=(nb,),
        in_specs=[
            pl.BlockSpec((bt, d), lambda i: (i, 0)),
            pl.BlockSpec((d, e), lambda i: (0, 0)),
        ],
        out_specs=[
            pl.BlockSpec((bt, 1), lambda i: (i, 0)),
            pl.BlockSpec((8, e), lambda i: (0, 0)),
            pl.BlockSpec((bt, d // 2), lambda i: (i, 0)),
        ],
        out_shape=[
            jax.ShapeDtypeStruct((t, 1), jnp.int32),
            jax.ShapeDtypeStruct((8, e), jnp.int32),
            jax.ShapeDtypeStruct((t, d // 2), jnp.int32),
        ],
        scratch_shapes=[pltpu.VMEM((1, e), jnp.float32)],
        compiler_params=pltpu.CompilerParams(
            dimension_semantics=("arbitrary",)),
    )(x, w_router)


def _gemm_body(d2, bo_ref, eo_ref, vld_ref, xs_ref, wi_ref, wo_ref, y_ref):
    g = pl.program_id(0)

    @pl.when(vld_ref[g] > 0)
    def _():
        u = jax.lax.bitcast_convert_type(xs_ref[...], jnp.uint32)
        x_lo = jax.lax.bitcast_convert_type(u << jnp.uint32(16),
                                            jnp.float32)
        x_hi = jax.lax.bitcast_convert_type(
            u & jnp.uint32(0xFFFF0000), jnp.float32)
        h = jnp.maximum(
            jnp.dot(x_lo, wi_ref[0, :d2], preferred_element_type=jnp.float32)
            + jnp.dot(x_hi, wi_ref[0, d2:], preferred_element_type=jnp.float32),
            0.0)
        y_ref[...] = jnp.dot(h, wo_ref[0], preferred_element_type=jnp.float32)


def _grouped_gemm(xs, wi, wo, bo, eo, vld, bt):
    t_pad, d2 = xs.shape
    e, d, f = wi.shape
    g = bo.shape[0]
    grid_spec = pltpu.PrefetchScalarGridSpec(
        num_scalar_prefetch=3,
        grid=(g,),
        in_specs=[
            pl.BlockSpec((bt, d2), lambda i, bo, eo, vld: (bo[i], 0)),
            pl.BlockSpec((1, d, f), lambda i, bo, eo, vld: (eo[i], 0, 0)),
            pl.BlockSpec((1, f, d), lambda i, bo, eo, vld: (eo[i], 0, 0)),
        ],
        out_specs=pl.BlockSpec((bt, d), lambda i, bo, eo, vld: (bo[i], 0)),
    )
    return pl.pallas_call(
        functools.partial(_gemm_body, d2),
        grid_spec=grid_spec,
        out_shape=jax.ShapeDtypeStruct((t_pad, d), jnp.float32),
        compiler_params=pltpu.CompilerParams(
            dimension_semantics=("parallel",)),
    )(bo, eo, vld, xs, wi, wo)


def _make_dispatch(t, t_pad, d2, e, nw, nc):
    p = t // nw
    c = min(64, p)
    nch = p // c
    log2e = e.bit_length() - 1
    mesh = plsc.VectorSubcoreMesh(core_axis_name="c", subcore_axis_name="s")

    @functools.partial(
        pl.kernel, mesh=mesh,
        out_type=[
            jax.ShapeDtypeStruct((t_pad, d2), jnp.int32),
            jax.ShapeDtypeStruct((t // c, c), jnp.int32),
        ],
        scratch_types=[
            pltpu.VMEM((e,), jnp.int32),
            pltpu.VMEM((p,), jnp.int32),
            pltpu.VMEM((nch, c), jnp.int32),
            pltpu.VMEM((2, c, d2), jnp.int32),
            pltpu.SemaphoreType.DMA,
            pltpu.SemaphoreType.DMA,
            pltpu.SemaphoreType.DMA,
            pltpu.SemaphoreType.DMA,
        ],
        compiler_params=pltpu.CompilerParams(needs_layout_passes=False),
    )
    def dispatch(enc_hbm, offs_hbm, xsc_hbm, xs_hbm, dest_hbm,
                 offs_v, enc_v, dest_v, rows2, si0, si1, so0, so1):
        wid = lax.axis_index("s") * nc + lax.axis_index("c")
        base = wid * p
        pltpu.sync_copy(offs_hbm, offs_v)
        pltpu.sync_copy(enc_hbm.at[pl.ds(base, p)], enc_v)
        per_row = c // 16
        for i in range(p // 16):
            enc16 = enc_v[pl.ds(i * 16, 16)]
            e16 = jnp.bitwise_and(enc16, e - 1)
            r16 = jax.lax.shift_right_logical(enc16, log2e)
            o16 = plsc.load_gather(offs_v, [e16])
            dest_v[i // per_row, pl.ds((i % per_row) * 16, 16)] = o16 + r16
        pltpu.sync_copy(dest_v, dest_hbm.at[pl.ds(wid * nch, nch)])

        sin = (si0, si1)
        sout = (so0, so1)
        hin = [None, None]
        hout = [None, None]
        hin[0] = pltpu.async_copy(xsc_hbm.at[pl.ds(base, c)], rows2.at[0],
                                  sin[0])
        for k in range(nch):
            b = k % 2
            hin[b].wait()
            if k >= 1:
                hout[1 - b].wait()
            if k + 1 < nch:
                hin[1 - b] = pltpu.async_copy(
                    xsc_hbm.at[pl.ds(base + (k + 1) * c, c)],
                    rows2.at[1 - b], sin[1 - b])
            hout[b] = pltpu.async_copy(rows2.at[b], xs_hbm.at[dest_v.at[k]],
                                       sout[b])
        hout[(nch - 1) % 2].wait()

    return dispatch


def _make_unsort(t, t_pad, d, nw, nc):
    p = t // nw
    c = min(64, p)
    nch = p // c
    mesh = plsc.VectorSubcoreMesh(core_axis_name="c", subcore_axis_name="s")

    @functools.partial(
        pl.kernel, mesh=mesh,
        out_type=jax.ShapeDtypeStruct((t, d), jnp.float32),
        scratch_types=[
            pltpu.VMEM((nch, c), jnp.int32),
            pltpu.VMEM((2, c, d), jnp.float32),
            pltpu.SemaphoreType.DMA,
            pltpu.SemaphoreType.DMA,
            pltpu.SemaphoreType.DMA,
            pltpu.SemaphoreType.DMA,
        ],
        compiler_params=pltpu.CompilerParams(needs_layout_passes=False),
    )
    def unsort(ys_hbm, dest_hbm, out_hbm, dest_v, rows2, si0, si1, so0, so1):
        wid = lax.axis_index("s") * nc + lax.axis_index("c")
        base = wid * p
        pltpu.sync_copy(dest_hbm.at[pl.ds(wid * nch, nch)], dest_v)

        sin = (si0, si1)
        sout = (so0, so1)
        hin = [None, None]
        hout = [None, None]
        hin[0] = pltpu.async_copy(ys_hbm.at[dest_v.at[0]], rows2.at[0],
                                  sin[0])
        for k in range(nch):
            b = k % 2
            hin[b].wait()
            if k >= 1:
                hout[1 - b].wait()
            if k + 1 < nch:
                hin[1 - b] = pltpu.async_copy(
                    ys_hbm.at[dest_v.at[k + 1]], rows2.at[1 - b],
                    sin[1 - b])
            hout[b] = pltpu.async_copy(rows2.at[b],
                                       out_hbm.at[pl.ds(base + k * c, c)],
                                       sout[b])
        hout[(nch - 1) % 2].wait()

    return unsort


def kernel(hidden_states, W_router, wi, wo):
    b, s, d = hidden_states.shape
    e = W_router.shape[1]
    f = wi.shape[2]
    t = b * s
    x = hidden_states.reshape(t, d)

    bt1 = 1024
    bt = 576
    g = -(-t // bt) + e
    t_pad = g * bt

    enc2, counts8, xsc = _router(x, W_router, bt1)
    counts = counts8[0]

    nblk = (counts + (bt - 1)) // bt
    cumblk = jnp.cumsum(nblk)
    nused = cumblk[-1]
    astart = ((cumblk - nblk) * bt).astype(jnp.int32)
    gi = jnp.arange(g, dtype=jnp.int32)
    eo = jnp.searchsorted(cumblk, gi, side="right").astype(jnp.int32)
    valid = gi < nused
    eo_f = jnp.where(valid, jnp.clip(eo, 0, e - 1), e - 1).astype(jnp.int32)
    bo = jnp.where(valid, gi, g - 1).astype(jnp.int32)
    vld = valid.astype(jnp.int32)

    info = plsc.get_sparse_core_info()
    nc, ns = info.num_cores, info.num_subcores
    nw = nc * ns

    dispatch = _make_dispatch(t, t_pad, d // 2, e, nw, nc)
    xs, dest = dispatch(enc2.reshape(t), astart, xsc)

    ys = _grouped_gemm(xs, wi, wo, bo, eo_f, vld, bt)

    unsort = _make_unsort(t, t_pad, d, nw, nc)
    out = unsort(ys, dest)
    return out.reshape(b, s, d)
